# baseline (speedup 1.0000x reference)
import jax
import jax.numpy as jnp
from jax import lax
from jax.experimental import pallas as pl
from jax.experimental.pallas import tpu as pltpu
from jax.experimental.pallas import tpu_sc as plsc

_N = 10000
_E = 320000
_F = 128
_NS = 16
_CHUNK = 64
_NCH = 320
_EPT = _CHUNK * _NCH
_EPAD = _NS * _EPT
_NPAD = 10240
_RPT = _NPAD // _NS
_RING = 4
_BLKCH = 16
_NBLK = _NCH // _BLKCH
_GTR = lax.GatherDimensionNumbers(offset_dims=(), collapsed_slice_dims=(0,),
                                  start_index_map=(0,))


def _spmm_body(u_hbm, src_hbm, dst_hbm, w_hbm, out_hbm,
               src_b, dst_b, w_b, gbuf, acc, gsem, ssem):
    s = lax.axis_index("s")

    zero16 = jnp.zeros((16,), jnp.float32)

    def zrow(r, carry):
        for f in range(_F // 16):
            gbuf[0, r, pl.ds(f * 16, 16)] = zero16
        return carry

    lax.fori_loop(0, _CHUNK, zrow, 0)
    base = s * _RPT
    for k in range(_RPT // _CHUNK):
        pltpu.sync_copy(gbuf.at[0], acc.at[pl.ds(base + k * _CHUNK, _CHUNK)])
    plsc.subcore_barrier()

    pltpu.sync_copy(src_hbm.at[s, pl.ds(0, _BLKCH)], src_b.at[0])
    for r in range(2):
        pltpu.async_copy(u_hbm.at[src_b.at[0, r]], gbuf.at[r], gsem.at[r])

    def chunk_body(j, carry):
        b = lax.rem(j, _RING)
        jm = lax.rem(j, _BLKCH)
        m = lax.div(j, _BLKCH)
        par = lax.rem(m, 2)

        @pl.when(jm == 0)
        def _():
            @pl.when(m < _NBLK - 1)
            def _():
                pltpu.sync_copy(
                    src_hbm.at[s, pl.ds((m + 1) * _BLKCH, _BLKCH)],
                    src_b.at[lax.rem(m + 1, 2)])
            pltpu.sync_copy(dst_hbm.at[s, pl.ds(m * _BLKCH, _BLKCH)],
                            dst_b.at[par])
            pltpu.sync_copy(w_hbm.at[s, pl.ds(m * _BLKCH * _CHUNK,
                                              _BLKCH * _CHUNK)], w_b)

        pltpu.make_async_copy(u_hbm.at[src_b.at[par, jm]], gbuf.at[b],
                              gsem.at[b]).wait()

        def grp(g, carry2):
            w16 = w_b[pl.ds(jm * _CHUNK + g * 16, 16)]
            for e in range(16):
                wb = lax.gather(
                    w16, jnp.full((16, 1), e, jnp.int32),
                    _GTR, slice_sizes=(1,),
                    mode=lax.GatherScatterMode.PROMISE_IN_BOUNDS)
                row = g * 16 + e
                for f in range(_F // 16):
                    sl = pl.ds(f * 16, 16)
                    gbuf[b, row, sl] = gbuf[b, row, sl] * wb
            return carry2

        lax.fori_loop(0, _CHUNK // 16, grp, 0)

        pltpu.async_copy(gbuf.at[b], acc.at[dst_b.at[par, jm]],
                         ssem.at[b], add=True)

        @pl.when(j >= 2)
        def _():
            bp = lax.rem(j - 2, _RING)
            pltpu.make_async_copy(gbuf.at[bp], acc.at[dst_b.at[par, jm]],
                                  ssem.at[bp]).wait()

            @pl.when(j + 2 < _NCH)
            def _():
                jn = j + 2
                pltpu.async_copy(
                    u_hbm.at[src_b.at[lax.rem(lax.div(jn, _BLKCH), 2),
                                      lax.rem(jn, _BLKCH)]],
                    gbuf.at[bp], gsem.at[bp])

        @pl.when(j < 2)
        def _():
            jn = j + 2
            pltpu.async_copy(
                u_hbm.at[src_b.at[lax.rem(lax.div(jn, _BLKCH), 2),
                                  lax.rem(jn, _BLKCH)]],
                gbuf.at[lax.rem(jn, _RING)], gsem.at[lax.rem(jn, _RING)])

        return carry

    lax.fori_loop(0, _NCH, chunk_body, 0)

    for t in ((_NCH - 2) % _RING, (_NCH - 1) % _RING):
        pltpu.make_async_copy(gbuf.at[t], acc.at[dst_b.at[0, 0]],
                              ssem.at[t]).wait()
    plsc.subcore_barrier()

    pltpu.sync_copy(acc.at[pl.ds(base, _RPT)],
                    out_hbm.at[pl.ds(base, _RPT)])


_spmm = pl.kernel(
    _spmm_body,
    out_type=jax.ShapeDtypeStruct((_NPAD, _F), jnp.float32),
    mesh=plsc.VectorSubcoreMesh(core_axis_name="c", subcore_axis_name="s",
                                num_cores=1),
    scratch_types=[
        pltpu.VMEM((2, _BLKCH, _CHUNK), jnp.int32),
        pltpu.VMEM((2, _BLKCH, _CHUNK), jnp.int32),
        pltpu.VMEM((_BLKCH * _CHUNK,), jnp.float32),
        pltpu.VMEM((_RING, _CHUNK, _F), jnp.float32),
        pltpu.VMEM_SHARED((_NPAD, _F), jnp.float32),
        pltpu.SemaphoreType.DMA((_RING,)),
        pltpu.SemaphoreType.DMA((_RING,)),
    ],
)

_BLK = 1000


def _mm_body(x_ref, w_ref, o_ref):
    o_ref[...] = jnp.dot(x_ref[...], w_ref[...],
                         preferred_element_type=jnp.float32)


def _fuse_mm_body(p_ref, b_ref, w_ref, o_ref):
    h = jnp.maximum(p_ref[...] + b_ref[...], 0.0)
    o_ref[...] = jnp.dot(h, w_ref[...], preferred_element_type=jnp.float32)


def _relu_body(p_ref, b_ref, o_ref):
    o_ref[...] = jnp.maximum(p_ref[...] + b_ref[...], 0.0)


def _mm_bias_body(p_ref, w_ref, b_ref, o_ref):
    o_ref[...] = jnp.dot(p_ref[...], w_ref[...],
                         preferred_element_type=jnp.float32) + b_ref[...]


def _lsm_body(x_ref, o_ref):
    x = x_ref[...]
    m = jnp.max(x, axis=0, keepdims=True)
    lse = m + jnp.log(jnp.sum(jnp.exp(x - m), axis=0, keepdims=True))
    o_ref[...] = x - lse


def _tc_mm(x, w):
    return pl.pallas_call(
        _mm_body,
        grid=(_N // _BLK,),
        in_specs=[pl.BlockSpec((_BLK, _F), lambda i: (i, 0)),
                  pl.BlockSpec((_F, _F), lambda i: (0, 0))],
        out_specs=pl.BlockSpec((_BLK, _F), lambda i: (i, 0)),
        out_shape=jax.ShapeDtypeStruct((_N, _F), jnp.float32),
    )(x, w)


def _tc_fuse_mm(p, b, w):
    return pl.pallas_call(
        _fuse_mm_body,
        grid=(_N // _BLK,),
        in_specs=[pl.BlockSpec((_BLK, _F), lambda i: (i, 0)),
                  pl.BlockSpec((_F,), lambda i: (0,)),
                  pl.BlockSpec((_F, _F), lambda i: (0, 0))],
        out_specs=pl.BlockSpec((_BLK, _F), lambda i: (i, 0)),
        out_shape=jax.ShapeDtypeStruct((_N, _F), jnp.float32),
    )(p, b, w)


def _tc_relu(p, b):
    return pl.pallas_call(
        _relu_body,
        grid=(_N // _BLK,),
        in_specs=[pl.BlockSpec((_BLK, _F), lambda i: (i, 0)),
                  pl.BlockSpec((_F,), lambda i: (0,))],
        out_specs=pl.BlockSpec((_BLK, _F), lambda i: (i, 0)),
        out_shape=jax.ShapeDtypeStruct((_N, _F), jnp.float32),
    )(p, b)


def _tc_mm_bias(p, w, b):
    nout = w.shape[1]
    return pl.pallas_call(
        _mm_bias_body,
        grid=(_N // _BLK,),
        in_specs=[pl.BlockSpec((_BLK, _F), lambda i: (i, 0)),
                  pl.BlockSpec((_F, nout), lambda i: (0, 0)),
                  pl.BlockSpec((nout,), lambda i: (0,))],
        out_specs=pl.BlockSpec((_BLK, nout), lambda i: (i, 0)),
        out_shape=jax.ShapeDtypeStruct((_N, nout), jnp.float32),
    )(p, w, b)


def _tc_lsm(x):
    nout = x.shape[1]
    return pl.pallas_call(
        _lsm_body,
        in_specs=[pl.BlockSpec((_N, nout), lambda: (0, 0))],
        out_specs=pl.BlockSpec((_N, nout), lambda: (0, 0)),
        out_shape=jax.ShapeDtypeStruct((_N, nout), jnp.float32),
    )(x)


def kernel(features, edge_index, edge_weight, W1, b1, W2, b2, W3, b3,
           W4, b4, W5, b5):
    pad = _EPAD - _E
    src = jnp.concatenate([edge_index[0], jnp.zeros((pad,), jnp.int32)])
    dst = jnp.concatenate([edge_index[1], jnp.zeros((pad,), jnp.int32)])
    w = jnp.concatenate([edge_weight, jnp.zeros((pad,), jnp.float32)])
    src3 = src.reshape(_NS, _NCH, _CHUNK)
    dst3 = dst.reshape(_NS, _NCH, _CHUNK)
    w2 = w.reshape(_NS, _EPT)

    def _sp(u):
        return _spmm(u, src3, dst3, w2)[:_N]

    u = _tc_mm(features, W1)
    for (b, W) in ((b1, W2), (b2, W3), (b3, W4)):
        p = _sp(u)
        u = _tc_fuse_mm(p, b, W)
    p = _sp(u)
    h4 = _tc_relu(p, b4)
    q = _sp(h4)
    logits = _tc_mm_bias(q, W5, b5)
    return _tc_lsm(logits)
